# baseline (speedup 1.0000x reference)
import functools

import jax
import jax.numpy as jnp
import numpy as np
from jax import lax
from jax.experimental import pallas as pl
from jax.experimental.pallas import tpu as pltpu
from jax.experimental.pallas import tpu_sc as plsc

IN_MUL = 16
OUT_MUL = 16
NEF = 128
WNUM = IN_MUL * OUT_MUL

NC = 2
NS = 16
NW = NC * NS
CHUNK = 128
GROUP = 8
RING = GROUP * CHUNK


def _tile_split(e):
    assert e % CHUNK == 0
    total_chunks = e // CHUNK
    mx = -(-total_chunks // NW)
    assert mx % GROUP == 0
    return total_chunks, mx


def _gather_kernel(e):
    total_chunks, mx = _tile_split(e)
    per_tile = mx * CHUNK
    last = total_chunks - (NW - 1) * mx
    assert last > 0
    mesh = plsc.VectorSubcoreMesh(core_axis_name="c", subcore_axis_name="s")

    @functools.partial(
        pl.kernel,
        out_type=jax.ShapeDtypeStruct((e * IN_MUL,), jnp.float32),
        mesh=mesh,
        compiler_params=pltpu.CompilerParams(use_tc_tiling_on_sc=False, needs_layout_passes=False),
        scratch_types=[
            pltpu.VMEM((per_tile,), jnp.int32),
            pltpu.VMEM((2 * RING, IN_MUL), jnp.float32),
            pltpu.VMEM((IN_MUL * per_tile,), jnp.float32),
            pltpu.SemaphoreType.DMA,
        ],
    )
    def gather_k(table_hbm, src_hbm, out_hbm, idx_v, ring_v, t_v, sem):
        c = lax.axis_index("c")
        s = lax.axis_index("s")
        wid = s * NC + c
        base = wid * per_tile

        @pl.when(wid < NW - 1)
        def _stage_full():
            pltpu.sync_copy(src_hbm.at[pl.ds(base, per_tile)], idx_v)

        @pl.when(wid == NW - 1)
        def _stage_last():
            pltpu.sync_copy(src_hbm.at[pl.ds(base, last * CHUNK)],
                            idx_v.at[pl.ds(0, last * CHUNK)])

        iota = lax.iota(jnp.int32, 16)
        iota_128 = iota * 128
        n_groups = mx // GROUP

        def _fire(g0, half):
            for g in range(g0, g0 + GROUP):
                @pl.when(wid * mx + g < total_chunks)
                def _f(g=g):
                    pltpu.async_copy(
                        table_hbm.at[idx_v.at[pl.ds(g * CHUNK, CHUNK)]],
                        ring_v.at[pl.ds(half * RING + (g - g0) * CHUNK,
                                        CHUNK)],
                        sem,
                    )

        def _drain(g0, half):
            for g in range(g0, g0 + GROUP):
                @pl.when(wid * mx + g < total_chunks)
                def _d(g=g):
                    pltpu.make_async_copy(
                        table_hbm.at[idx_v.at[pl.ds(g * CHUNK, CHUNK)]],
                        ring_v.at[pl.ds(half * RING + (g - g0) * CHUNK,
                                        CHUNK)],
                        sem,
                    ).wait()

        _fire(0, 0)
        for gi in range(n_groups):
            g0 = gi * GROUP
            half = gi % 2
            _drain(g0, half)
            if gi + 1 < n_groups:
                _fire(g0 + GROUP, 1 - half)

            def _tbody(l, g0=g0, half=half):
                v = ring_v[half * RING + l]
                le = g0 * CHUNK + l
                off = (le >> 7) * (16 * 128) + (le & 127)
                plsc.store_scatter(t_v, [iota_128 + off], v)

            plsc.parallel_loop(0, RING, unroll=8)(_tbody)

        @pl.when(wid < NW - 1)
        def _out_full():
            pltpu.sync_copy(t_v,
                            out_hbm.at[pl.ds(base * IN_MUL,
                                             per_tile * IN_MUL)])

        @pl.when(wid == NW - 1)
        def _out_last():
            pltpu.sync_copy(t_v.at[pl.ds(0, last * CHUNK * IN_MUL)],
                            out_hbm.at[pl.ds(base * IN_MUL,
                                             last * CHUNK * IN_MUL)])

    return gather_k


def _scatter_kernel(e, n_acc):
    total_chunks, mx = _tile_split(e)
    per_tile = mx * CHUNK
    last = total_chunks - (NW - 1) * mx
    rows_per_tile = n_acc // NS
    mesh = plsc.VectorSubcoreMesh(core_axis_name="c", subcore_axis_name="s")

    @functools.partial(
        pl.kernel,
        out_type=jax.ShapeDtypeStruct((NC, n_acc, OUT_MUL), jnp.float32),
        mesh=mesh,
        compiler_params=pltpu.CompilerParams(use_tc_tiling_on_sc=False, needs_layout_passes=False),
        scratch_types=[
            pltpu.VMEM((mx, CHUNK), jnp.int32),
            pltpu.VMEM((2 * RING, OUT_MUL), jnp.float32),
            pltpu.VMEM((OUT_MUL * per_tile,), jnp.float32),
            pltpu.VMEM_SHARED((n_acc, OUT_MUL), jnp.float32),
            pltpu.SemaphoreType.DMA,
            pltpu.SemaphoreType.DMA,
        ],
    )
    def scatter_k(tp_hbm, dst_hbm, zeros_hbm, out_hbm, idx_v, ring_v, t_v,
                  acc_sh, sem, sem2):
        c = lax.axis_index("c")
        s = lax.axis_index("s")
        wid = s * NC + c
        base = wid * per_tile

        @pl.when(s == 0)
        def _init():
            pltpu.sync_copy(zeros_hbm, acc_sh)

        for g in range(mx):
            @pl.when(wid * mx + g < total_chunks)
            def _idx(g=g):
                pltpu.async_copy(
                    dst_hbm.at[pl.ds(base + g * CHUNK, CHUNK)],
                    idx_v.at[g], sem2)

        @pl.when(wid < NW - 1)
        def _stage_full():
            pltpu.async_copy(
                tp_hbm.at[pl.ds(base * OUT_MUL, per_tile * OUT_MUL)],
                t_v, sem)

        @pl.when(wid == NW - 1)
        def _stage_last():
            pltpu.async_copy(
                tp_hbm.at[pl.ds(base * OUT_MUL, last * CHUNK * OUT_MUL)],
                t_v.at[pl.ds(0, last * CHUNK * OUT_MUL)], sem)

        for g in range(mx):
            @pl.when(wid * mx + g < total_chunks)
            def _idxw(g=g):
                pltpu.make_async_copy(
                    dst_hbm.at[pl.ds(base + g * CHUNK, CHUNK)],
                    idx_v.at[g], sem2).wait()

        @pl.when(wid < NW - 1)
        def _wait_full():
            pltpu.make_async_copy(
                tp_hbm.at[pl.ds(base * OUT_MUL, per_tile * OUT_MUL)],
                t_v, sem).wait()

        @pl.when(wid == NW - 1)
        def _wait_last():
            pltpu.make_async_copy(
                tp_hbm.at[pl.ds(base * OUT_MUL, last * CHUNK * OUT_MUL)],
                t_v.at[pl.ds(0, last * CHUNK * OUT_MUL)], sem).wait()

        plsc.subcore_barrier()
        iota = lax.iota(jnp.int32, 16)
        iota_128 = iota * 128
        n_groups = mx // GROUP

        def _transpose(g0, half):
            def _tbody(l, g0=g0, half=half):
                le = g0 * CHUNK + l
                off = (le >> 7) * (16 * 128) + (le & 127)
                v = plsc.load_gather(t_v, [iota_128 + off])
                ring_v[half * RING + l] = v

            plsc.parallel_loop(0, RING, unroll=8)(_tbody)

        def _fire_adds(g0, half):
            for g in range(g0, g0 + GROUP):
                @pl.when(wid * mx + g < total_chunks)
                def _a(g=g):
                    pltpu.async_copy(
                        ring_v.at[pl.ds(half * RING + (g - g0) * CHUNK,
                                        CHUNK)],
                        acc_sh.at[idx_v.at[g]],
                        sem, add=True,
                    )

        def _drain_adds(g0, half):
            for g in range(g0, g0 + GROUP):
                @pl.when(wid * mx + g < total_chunks)
                def _w(g=g):
                    pltpu.make_async_copy(
                        ring_v.at[pl.ds(half * RING + (g - g0) * CHUNK,
                                        CHUNK)],
                        acc_sh.at[idx_v.at[g]],
                        sem,
                    ).wait()

        _transpose(0, 0)
        for gi in range(n_groups):
            g0 = gi * GROUP
            half = gi % 2
            _fire_adds(g0, half)
            if gi + 1 < n_groups:
                _transpose(g0 + GROUP, 1 - half)
            _drain_adds(g0, half)
        plsc.subcore_barrier()
        pltpu.sync_copy(
            acc_sh.at[pl.ds(s * rows_per_tile, rows_per_tile)],
            out_hbm.at[c, pl.ds(s * rows_per_tile, rows_per_tile)],
        )

    return scatter_k


def _tc_edge_body(ea_ref, x1t_ref, sh_ref, w1_ref, b1_ref, w2_ref, b2_ref,
                  r_ref, s_ref, tpt_ref):
    t = ea_ref.shape[0]
    ea = ea_ref[...].astype(jnp.bfloat16)
    h = jnp.dot(ea, w1_ref[...], preferred_element_type=jnp.float32)
    h = jnp.maximum(h + b1_ref[...], 0.0).astype(jnp.bfloat16)
    w = jnp.dot(h, w2_ref[...], preferred_element_type=jnp.float32) + b2_ref[...]
    tb = t // 128
    x1e = jnp.swapaxes(x1t_ref[...], 1, 2).reshape(t, IN_MUL)
    x1r = jnp.dot(x1e.astype(jnp.bfloat16), r_ref[...],
                  preferred_element_type=jnp.float32)
    prod = (x1r * w).astype(jnp.bfloat16)
    tp = jnp.dot(prod, s_ref[...], preferred_element_type=jnp.float32)
    tp3 = jnp.swapaxes(tp.reshape(tb, 128, OUT_MUL), 1, 2)
    tpt_ref[...] = tp3 * sh_ref[...].reshape(tb, 1, 128)


def _bn_body(p0_ref, p1_ref, na_ref, w_ref, b_ref, m_ref, out_ref):
    n = p0_ref.shape[0] * 8
    s = p0_ref[...] + p1_ref[...] + na_ref[...]
    sums = jnp.sum(s, axis=0, keepdims=True)
    mean = jnp.dot(sums, m_ref[...],
                   preferred_element_type=jnp.float32) * (1.0 / n)
    cent = s - mean
    vsum = jnp.sum(cent * cent, axis=0, keepdims=True)
    var = jnp.dot(vsum, m_ref[...],
                  preferred_element_type=jnp.float32) * (1.0 / n)
    out_ref[...] = cent * lax.rsqrt(var + 1e-5) * w_ref[...] + b_ref[...]


def kernel(node_attr, edge_index, edge_attr, edge_sh, W1, b1, W2, b2,
           bn_weight, bn_bias):
    n = node_attr.shape[0]
    e = edge_attr.shape[0]

    src = edge_index[0]
    dst = edge_index[1]

    x1t = _gather_kernel(e)(node_attr, src)

    alpha = 1.0 / np.sqrt(IN_MUL * 1)
    i_idx = np.arange(WNUM) // OUT_MUL
    k_idx = np.arange(WNUM) % OUT_MUL
    R = jnp.asarray((i_idx[None, :] == np.arange(IN_MUL)[:, None])
                    .astype(np.float32) * alpha).astype(jnp.bfloat16)
    S = jnp.asarray((k_idx[:, None] == np.arange(OUT_MUL)[None, :])
                    .astype(np.float32)).astype(jnp.bfloat16)
    T = 3200
    while e % T:
        T //= 2
    grid = (e // T,)
    ec = e // 128
    tc = T // 128
    sh3 = edge_sh.reshape(ec, 1, 128)
    x1t3 = x1t.reshape(ec, IN_MUL, 128)
    tpt3 = pl.pallas_call(
        _tc_edge_body,
        grid=grid,
        in_specs=[
            pl.BlockSpec((T, NEF), lambda i: (i, 0)),
            pl.BlockSpec((tc, IN_MUL, 128), lambda i: (i, 0, 0)),
            pl.BlockSpec((tc, 1, 128), lambda i: (i, 0, 0)),
            pl.BlockSpec((NEF, NEF), lambda i: (0, 0)),
            pl.BlockSpec((1, NEF), lambda i: (0, 0)),
            pl.BlockSpec((NEF, WNUM), lambda i: (0, 0)),
            pl.BlockSpec((1, WNUM), lambda i: (0, 0)),
            pl.BlockSpec((IN_MUL, WNUM), lambda i: (0, 0)),
            pl.BlockSpec((WNUM, OUT_MUL), lambda i: (0, 0)),
        ],
        out_specs=pl.BlockSpec((tc, OUT_MUL, 128), lambda i: (i, 0, 0)),
        out_shape=jax.ShapeDtypeStruct((ec, OUT_MUL, 128), jnp.float32),
    )(edge_attr, x1t3, sh3,
      W1.astype(jnp.bfloat16), b1.reshape(1, NEF),
      W2.astype(jnp.bfloat16), b2.reshape(1, WNUM), R, S)
    tpt = tpt3.reshape(e * OUT_MUL)

    n_acc = ((n + (NS * 8) - 1) // (NS * 8)) * (NS * 8)
    zeros = jnp.zeros((n_acc, OUT_MUL), jnp.float32)
    partials = _scatter_kernel(e, n_acc)(tpt, dst, zeros)

    assert n * OUT_MUL % 128 == 0
    nr = n * OUT_MUL // 128
    pf = partials.reshape(2 * n_acc * OUT_MUL // 128, 128)
    p0p = pf[:nr]
    p1p = pf[n_acc * OUT_MUL // 128:n_acc * OUT_MUL // 128 + nr]
    nap = node_attr.reshape(nr, 128)
    lane = np.arange(128)
    M = jnp.asarray((lane[:, None] % OUT_MUL == lane[None, :] % OUT_MUL)
                    .astype(np.float32))
    wrow = jnp.tile(bn_weight, 128 // OUT_MUL).reshape(1, 128)
    brow = jnp.tile(bn_bias, 128 // OUT_MUL).reshape(1, 128)
    outp = pl.pallas_call(
        _bn_body,
        out_shape=jax.ShapeDtypeStruct((nr, 128), jnp.float32),
    )(p0p, p1p, nap, wrow, brow, M)
    return outp.reshape(n, OUT_MUL)


# speedup vs baseline: 1.0319x; 1.0319x over previous
import functools

import jax
import jax.numpy as jnp
import numpy as np
from jax import lax
from jax.experimental import pallas as pl
from jax.experimental.pallas import tpu as pltpu
from jax.experimental.pallas import tpu_sc as plsc

IN_MUL = 16
OUT_MUL = 16
NEF = 128
WNUM = IN_MUL * OUT_MUL

NC = 2
NS = 16
NW = NC * NS
CHUNK = 128
GROUP = 8
RING = GROUP * CHUNK


def _tile_split(e):
    assert e % CHUNK == 0
    total_chunks = e // CHUNK
    mx = -(-total_chunks // NW)
    assert mx % GROUP == 0
    return total_chunks, mx


def _gather_kernel(e):
    total_chunks, mx = _tile_split(e)
    per_tile = mx * CHUNK
    last = total_chunks - (NW - 1) * mx
    assert last > 0
    mesh = plsc.VectorSubcoreMesh(core_axis_name="c", subcore_axis_name="s")

    @functools.partial(
        pl.kernel,
        out_type=jax.ShapeDtypeStruct((e * IN_MUL,), jnp.float32),
        mesh=mesh,
        compiler_params=pltpu.CompilerParams(use_tc_tiling_on_sc=False, needs_layout_passes=False),
        scratch_types=[
            pltpu.VMEM((per_tile,), jnp.int32),
            pltpu.VMEM((2 * RING, IN_MUL), jnp.float32),
            pltpu.VMEM((IN_MUL * per_tile,), jnp.float32),
            pltpu.SemaphoreType.DMA,
        ],
    )
    def gather_k(table_hbm, src_hbm, out_hbm, idx_v, ring_v, t_v, sem):
        c = lax.axis_index("c")
        s = lax.axis_index("s")
        wid = s * NC + c
        base = wid * per_tile

        @pl.when(wid < NW - 1)
        def _stage_full():
            pltpu.sync_copy(src_hbm.at[pl.ds(base, per_tile)], idx_v)

        @pl.when(wid == NW - 1)
        def _stage_last():
            pltpu.sync_copy(src_hbm.at[pl.ds(base, last * CHUNK)],
                            idx_v.at[pl.ds(0, last * CHUNK)])

        iota = lax.iota(jnp.int32, 16)
        iota_128 = iota * 128
        n_groups = mx // GROUP

        def _fire(g0, half):
            for g in range(g0, g0 + GROUP):
                @pl.when(wid * mx + g < total_chunks)
                def _f(g=g):
                    pltpu.async_copy(
                        table_hbm.at[idx_v.at[pl.ds(g * CHUNK, CHUNK)]],
                        ring_v.at[pl.ds(half * RING + (g - g0) * CHUNK,
                                        CHUNK)],
                        sem,
                    )

        def _drain(g0, half):
            for g in range(g0, g0 + GROUP):
                @pl.when(wid * mx + g < total_chunks)
                def _d(g=g):
                    pltpu.make_async_copy(
                        table_hbm.at[idx_v.at[pl.ds(g * CHUNK, CHUNK)]],
                        ring_v.at[pl.ds(half * RING + (g - g0) * CHUNK,
                                        CHUNK)],
                        sem,
                    ).wait()

        _fire(0, 0)
        for gi in range(n_groups):
            g0 = gi * GROUP
            half = gi % 2
            _drain(g0, half)
            if gi + 1 < n_groups:
                _fire(g0 + GROUP, 1 - half)

            def _tbody(l, g0=g0, half=half):
                v = ring_v[half * RING + l]
                le = g0 * CHUNK + l
                off = (le >> 7) * (16 * 128) + (le & 127)
                plsc.store_scatter(t_v, [iota_128 + off], v)

            plsc.parallel_loop(0, RING, unroll=8)(_tbody)

        @pl.when(wid < NW - 1)
        def _out_full():
            pltpu.sync_copy(t_v,
                            out_hbm.at[pl.ds(base * IN_MUL,
                                             per_tile * IN_MUL)])

        @pl.when(wid == NW - 1)
        def _out_last():
            pltpu.sync_copy(t_v.at[pl.ds(0, last * CHUNK * IN_MUL)],
                            out_hbm.at[pl.ds(base * IN_MUL,
                                             last * CHUNK * IN_MUL)])

    return gather_k


def _scatter_kernel(e, n_acc):
    total_chunks, mx = _tile_split(e)
    per_tile = mx * CHUNK
    last = total_chunks - (NW - 1) * mx
    rows_per_tile = n_acc // NS
    mesh = plsc.VectorSubcoreMesh(core_axis_name="c", subcore_axis_name="s")

    @functools.partial(
        pl.kernel,
        out_type=jax.ShapeDtypeStruct((NC, n_acc, OUT_MUL), jnp.float32),
        mesh=mesh,
        compiler_params=pltpu.CompilerParams(use_tc_tiling_on_sc=False, needs_layout_passes=False),
        scratch_types=[
            pltpu.VMEM((mx, CHUNK), jnp.int32),
            pltpu.VMEM((2 * RING, OUT_MUL), jnp.float32),
            pltpu.VMEM((OUT_MUL * per_tile,), jnp.float32),
            pltpu.VMEM_SHARED((n_acc, OUT_MUL), jnp.float32),
            pltpu.SemaphoreType.DMA,
            pltpu.SemaphoreType.DMA,
        ],
    )
    def scatter_k(tp_hbm, dst_hbm, zeros_hbm, out_hbm, idx_v, ring_v, t_v,
                  acc_sh, sem, sem2):
        c = lax.axis_index("c")
        s = lax.axis_index("s")
        wid = s * NC + c
        base = wid * per_tile

        @pl.when(s == 0)
        def _init():
            pltpu.sync_copy(zeros_hbm, acc_sh)

        for g in range(mx):
            @pl.when(wid * mx + g < total_chunks)
            def _idx(g=g):
                pltpu.async_copy(
                    dst_hbm.at[pl.ds(base + g * CHUNK, CHUNK)],
                    idx_v.at[g], sem2)

        @pl.when(wid < NW - 1)
        def _stage_full():
            pltpu.async_copy(
                tp_hbm.at[pl.ds(base * OUT_MUL, per_tile * OUT_MUL)],
                t_v, sem)

        @pl.when(wid == NW - 1)
        def _stage_last():
            pltpu.async_copy(
                tp_hbm.at[pl.ds(base * OUT_MUL, last * CHUNK * OUT_MUL)],
                t_v.at[pl.ds(0, last * CHUNK * OUT_MUL)], sem)

        for g in range(mx):
            @pl.when(wid * mx + g < total_chunks)
            def _idxw(g=g):
                pltpu.make_async_copy(
                    dst_hbm.at[pl.ds(base + g * CHUNK, CHUNK)],
                    idx_v.at[g], sem2).wait()

        @pl.when(wid < NW - 1)
        def _wait_full():
            pltpu.make_async_copy(
                tp_hbm.at[pl.ds(base * OUT_MUL, per_tile * OUT_MUL)],
                t_v, sem).wait()

        @pl.when(wid == NW - 1)
        def _wait_last():
            pltpu.make_async_copy(
                tp_hbm.at[pl.ds(base * OUT_MUL, last * CHUNK * OUT_MUL)],
                t_v.at[pl.ds(0, last * CHUNK * OUT_MUL)], sem).wait()

        plsc.subcore_barrier()
        iota = lax.iota(jnp.int32, 16)
        iota_128 = iota * 128
        n_groups = mx // GROUP

        def _transpose(g0, half):
            def _tbody(l, g0=g0, half=half):
                le = g0 * CHUNK + l
                off = (le >> 7) * (16 * 128) + (le & 127)
                v = plsc.load_gather(t_v, [iota_128 + off])
                ring_v[half * RING + l] = v

            plsc.parallel_loop(0, RING, unroll=8)(_tbody)

        def _fire_adds(g0, half):
            for g in range(g0, g0 + GROUP):
                @pl.when(wid * mx + g < total_chunks)
                def _a(g=g):
                    pltpu.async_copy(
                        ring_v.at[pl.ds(half * RING + (g - g0) * CHUNK,
                                        CHUNK)],
                        acc_sh.at[idx_v.at[g]],
                        sem, add=True,
                    )

        def _drain_adds(g0, half):
            for g in range(g0, g0 + GROUP):
                @pl.when(wid * mx + g < total_chunks)
                def _w(g=g):
                    pltpu.make_async_copy(
                        ring_v.at[pl.ds(half * RING + (g - g0) * CHUNK,
                                        CHUNK)],
                        acc_sh.at[idx_v.at[g]],
                        sem,
                    ).wait()

        _transpose(0, 0)
        for gi in range(n_groups):
            g0 = gi * GROUP
            half = gi % 2
            _fire_adds(g0, half)
            if gi + 1 < n_groups:
                _transpose(g0 + GROUP, 1 - half)
            _drain_adds(g0, half)
        plsc.subcore_barrier()
        pltpu.sync_copy(
            acc_sh.at[pl.ds(s * rows_per_tile, rows_per_tile)],
            out_hbm.at[c, pl.ds(s * rows_per_tile, rows_per_tile)],
        )

    return scatter_k


def _tc_edge_body(ea_ref, x1t_ref, sh_ref, w1_ref, b1_ref, w2_ref, b2_ref,
                  r_ref, s_ref, tpt_ref):
    t = ea_ref.shape[0]
    ea = ea_ref[...].astype(jnp.bfloat16)
    h = jnp.dot(ea, w1_ref[...], preferred_element_type=jnp.float32)
    h = jnp.maximum(h + b1_ref[...], 0.0).astype(jnp.bfloat16)
    w = jnp.dot(h, w2_ref[...], preferred_element_type=jnp.float32) + b2_ref[...]
    tb = t // 128
    x1e = jnp.swapaxes(x1t_ref[...], 1, 2).reshape(t, IN_MUL)
    x1r = jnp.dot(x1e.astype(jnp.bfloat16), r_ref[...],
                  preferred_element_type=jnp.float32)
    prod = (x1r * w).astype(jnp.bfloat16)
    tp = jnp.dot(prod, s_ref[...], preferred_element_type=jnp.float32)
    tp3 = jnp.swapaxes(tp.reshape(tb, 128, OUT_MUL), 1, 2)
    tpt_ref[...] = tp3 * sh_ref[...].reshape(tb, 1, 128)


def _bn_body(p0_ref, p1_ref, na_ref, w_ref, b_ref, m_ref, out_ref):
    n = p0_ref.shape[0] * 8
    s = p0_ref[...] + p1_ref[...] + na_ref[...]
    sums = jnp.sum(s, axis=0, keepdims=True)
    mean = jnp.dot(sums, m_ref[...],
                   preferred_element_type=jnp.float32) * (1.0 / n)
    cent = s - mean
    vsum = jnp.sum(cent * cent, axis=0, keepdims=True)
    var = jnp.dot(vsum, m_ref[...],
                  preferred_element_type=jnp.float32) * (1.0 / n)
    out_ref[...] = cent * lax.rsqrt(var + 1e-5) * w_ref[...] + b_ref[...]


def kernel(node_attr, edge_index, edge_attr, edge_sh, W1, b1, W2, b2,
           bn_weight, bn_bias):
    n = node_attr.shape[0]
    e = edge_attr.shape[0]

    src = edge_index[0]
    dst = edge_index[1]

    x1t = _gather_kernel(e)(node_attr, src)

    alpha = 1.0 / np.sqrt(IN_MUL * 1)
    i_idx = np.arange(WNUM) // OUT_MUL
    k_idx = np.arange(WNUM) % OUT_MUL
    R = jnp.asarray((i_idx[None, :] == np.arange(IN_MUL)[:, None])
                    .astype(np.float32) * alpha).astype(jnp.bfloat16)
    S = jnp.asarray((k_idx[:, None] == np.arange(OUT_MUL)[None, :])
                    .astype(np.float32)).astype(jnp.bfloat16)
    T = 6400
    while e % T:
        T //= 2
    grid = (e // T,)
    ec = e // 128
    tc = T // 128
    sh3 = edge_sh.reshape(ec, 1, 128)
    x1t3 = x1t.reshape(ec, IN_MUL, 128)
    tpt3 = pl.pallas_call(
        _tc_edge_body,
        grid=grid,
        in_specs=[
            pl.BlockSpec((T, NEF), lambda i: (i, 0)),
            pl.BlockSpec((tc, IN_MUL, 128), lambda i: (i, 0, 0)),
            pl.BlockSpec((tc, 1, 128), lambda i: (i, 0, 0)),
            pl.BlockSpec((NEF, NEF), lambda i: (0, 0)),
            pl.BlockSpec((1, NEF), lambda i: (0, 0)),
            pl.BlockSpec((NEF, WNUM), lambda i: (0, 0)),
            pl.BlockSpec((1, WNUM), lambda i: (0, 0)),
            pl.BlockSpec((IN_MUL, WNUM), lambda i: (0, 0)),
            pl.BlockSpec((WNUM, OUT_MUL), lambda i: (0, 0)),
        ],
        out_specs=pl.BlockSpec((tc, OUT_MUL, 128), lambda i: (i, 0, 0)),
        out_shape=jax.ShapeDtypeStruct((ec, OUT_MUL, 128), jnp.float32),
    )(edge_attr, x1t3, sh3,
      W1.astype(jnp.bfloat16), b1.reshape(1, NEF),
      W2.astype(jnp.bfloat16), b2.reshape(1, WNUM), R, S)
    tpt = tpt3.reshape(e * OUT_MUL)

    n_acc = ((n + (NS * 8) - 1) // (NS * 8)) * (NS * 8)
    zeros = jnp.zeros((n_acc, OUT_MUL), jnp.float32)
    partials = _scatter_kernel(e, n_acc)(tpt, dst, zeros)

    assert n * OUT_MUL % 128 == 0
    nr = n * OUT_MUL // 128
    pf = partials.reshape(2 * n_acc * OUT_MUL // 128, 128)
    p0p = pf[:nr]
    p1p = pf[n_acc * OUT_MUL // 128:n_acc * OUT_MUL // 128 + nr]
    nap = node_attr.reshape(nr, 128)
    lane = np.arange(128)
    M = jnp.asarray((lane[:, None] % OUT_MUL == lane[None, :] % OUT_MUL)
                    .astype(np.float32))
    wrow = jnp.tile(bn_weight, 128 // OUT_MUL).reshape(1, 128)
    brow = jnp.tile(bn_bias, 128 // OUT_MUL).reshape(1, 128)
    outp = pl.pallas_call(
        _bn_body,
        out_shape=jax.ShapeDtypeStruct((nr, 128), jnp.float32),
    )(p0p, p1p, nap, wrow, brow, M)
    return outp.reshape(n, OUT_MUL)
